# baseline (speedup 1.0000x reference)
import functools

import jax
import jax.numpy as jnp
from jax.experimental import pallas as pl
from jax.experimental.pallas import tpu as pltpu

_HEAD_W = 8


def _block_body(T, HID, C, x_ref, wx_ref, bx_ref, whzr_ref, bhzr_ref,
                whh_ref, bhh_ref, wup_ref, bup_ref,
                wu8_ref, wp8_ref, bh8_ref, y_ref):
    f32 = jnp.float32
    bf16 = jnp.bfloat16
    wx = wx_ref[...]
    whzr = whzr_ref[...]
    whh = whh_ref[...]
    wup = wup_ref[...]
    wu8 = wu8_ref[...]
    wp8 = wp8_ref[...]
    bn = x_ref.shape[2]
    cw = bn // C
    hs = [jnp.zeros((cw, HID), f32) for _ in range(C)]
    for t in range(T):
        x = x_ref[t]
        for c in range(C):
            h = hs[c]
            xc = x[:, c * cw:(c + 1) * cw]
            xp = jax.lax.dot_general(xc, wx, (((0,), (0,)), ((), ())),
                                     preferred_element_type=f32) + bx_ref[...]
            h16 = h.astype(bf16)
            hzr = jnp.dot(h16, whzr, preferred_element_type=f32) + bhzr_ref[...]
            zr = 0.5 + 0.5 * jnp.tanh(xp[:, :2 * HID] + hzr)
            z = zr[:, :HID]
            r = zr[:, HID:]
            ht = jnp.tanh(xp[:, 2 * HID:]
                          + jnp.dot((h * r).astype(bf16), whh,
                                    preferred_element_type=f32)
                          + bhh_ref[...])
            h = ht + z * (h - ht)
            hs[c] = h
            h16 = h.astype(bf16)
            tq = jnp.tanh(jnp.dot(h16, wup, preferred_element_type=f32)
                          + bup_ref[...])
            hu = ((0.5 + 0.5 * tq[:, :HID]) * tq[:, HID:2 * HID]).astype(bf16)
            hp = ((0.5 + 0.5 * tq[:, 2 * HID:3 * HID]) * tq[:, 3 * HID:]).astype(bf16)
            y = (jax.lax.dot_general(wu8, hu, (((0,), (1,)), ((), ())),
                                     preferred_element_type=f32)
                 + jax.lax.dot_general(wp8, hp, (((0,), (1,)), ((), ())),
                                       preferred_element_type=f32))
            y_ref[t, :, c * cw:(c + 1) * cw] = y + bh8_ref[...]


@jax.jit
def kernel(X_seq, edge, params):
    del edge
    T, N, IN_F = X_seq.shape
    pb = params["backbone"]
    HID = pb["W_hz"].shape[0]
    f32 = jnp.float32

    wx = jnp.concatenate([pb["W_xz"], pb["W_xr"], pb["W_xh"]], axis=1)
    bx = jnp.concatenate([pb["b_xz"], pb["b_xr"], pb["b_xh"]])[None, :]
    whzr = jnp.concatenate([pb["W_hz"], pb["W_hr"]], axis=1)
    bhzr = jnp.concatenate([pb["b_hz"], pb["b_hr"]])[None, :]
    whh = pb["W_hh"]
    bhh = pb["b_hh"][None, :]

    def head_cell(p):
        w = jnp.concatenate([p["W_xz"], p["W_xh"]], axis=1)
        b = jnp.concatenate([p["b_xz"] + p["b_hz"], p["b_xh"] + p["b_hh"]])[None, :]
        return w, b

    wu, bu = head_cell(params["gru_u"])
    wp, bp = head_cell(params["gru_sp"])
    wup = jnp.concatenate([wu, wp], axis=1)
    bup = jnp.concatenate([bu, bp], axis=1)
    sxz = jnp.concatenate([jnp.full((2 * HID,), 0.5, f32),
                           jnp.ones((HID,), f32)])[None, :]
    wx = wx * sxz
    bx = bx * sxz
    whzr = whzr * 0.5
    bhzr = bhzr * 0.5
    sup = jnp.concatenate([jnp.full((HID,), -0.5, f32), jnp.ones((HID,), f32),
                           jnp.full((HID,), -0.5, f32), jnp.ones((HID,), f32)])[None, :]
    wup = wup * sup
    bup = bup * sup
    bf16 = jnp.bfloat16
    wx, whzr, whh, wup = (a.astype(bf16) for a in (wx, whzr, whh, wup))

    wu8 = jnp.zeros((HID, _HEAD_W), bf16).at[:, 0:3].set(params["W_hu"].astype(bf16))
    wp8 = (jnp.zeros((HID, _HEAD_W), bf16)
           .at[:, 3:4].set(params["W_hs"].astype(bf16))
           .at[:, 4:5].set(params["W_hp"].astype(bf16)))
    bh8 = (jnp.zeros((_HEAD_W,), f32)
           .at[0:3].set(params["b_hu"])
           .at[3].set(params["b_hs"][0])
           .at[4].set(params["b_hp"][0]))[:, None]

    bn = 2048
    xt = X_seq.transpose(0, 2, 1).astype(jnp.bfloat16)
    grid = pl.cdiv(N, bn)

    full = lambda a: pl.BlockSpec(a.shape, lambda i: (0,) * a.ndim)
    y = pl.pallas_call(
        functools.partial(_block_body, T, HID, 2),
        grid=(grid,),
        in_specs=[
            pl.BlockSpec((T, IN_F, bn), lambda i: (0, 0, i)),
            full(wx), full(bx), full(whzr), full(bhzr), full(whh), full(bhh),
            full(wup), full(bup),
            full(wu8), full(wp8), full(bh8),
        ],
        out_specs=pl.BlockSpec((T, _HEAD_W, bn), lambda i: (0, 0, i)),
        out_shape=jax.ShapeDtypeStruct((T, _HEAD_W, N), f32),
        compiler_params=pltpu.CompilerParams(
            dimension_semantics=("parallel",)),
    )(xt, wx, bx, whzr, bhzr, whh, bhh, wup, bup, wu8, wp8, bh8)

    out_u = y[:, 0:3, :].transpose(0, 2, 1)
    out_s = y[:, 3, :]
    out_p = y[:, 4, :]
    return (out_u, out_s, out_p)


# speedup vs baseline: 1.0919x; 1.0919x over previous
import functools

import jax
import jax.numpy as jnp
from jax.experimental import pallas as pl
from jax.experimental.pallas import tpu as pltpu

_HEAD_W = 8


def _block_body(T, HID, C, x_ref, wx_ref, bx_ref, whzr_ref, bhzr_ref,
                whh_ref, bhh_ref, wup_ref, bup_ref,
                wu8_ref, wp8_ref, bh8_ref, y_ref):
    f32 = jnp.float32
    bf16 = jnp.bfloat16
    wx = wx_ref[...]
    whzr = whzr_ref[...]
    whh = whh_ref[...]
    wup = wup_ref[...]
    wu8 = wu8_ref[...]
    wp8 = wp8_ref[...]
    bn = x_ref.shape[2]
    cw = bn // C
    hs = [jnp.zeros((cw, HID), f32) for _ in range(C)]
    for t in range(T):
        x = x_ref[t]
        for c in range(C):
            h = hs[c]
            xc = x[:, c * cw:(c + 1) * cw]
            xp = jax.lax.dot_general(xc, wx, (((0,), (0,)), ((), ())),
                                     preferred_element_type=f32) + bx_ref[...]
            h16 = h.astype(bf16)
            hzr = jnp.dot(h16, whzr, preferred_element_type=f32) + bhzr_ref[...]
            zr = 0.5 + 0.5 * jnp.tanh(xp[:, :2 * HID] + hzr)
            z = zr[:, :HID]
            r = zr[:, HID:]
            ht = jnp.tanh(xp[:, 2 * HID:]
                          + jnp.dot((h * r).astype(bf16), whh,
                                    preferred_element_type=f32)
                          + bhh_ref[...])
            h = ht + z * (h - ht)
            hs[c] = h
            h16 = h.astype(bf16)
            tq = jnp.tanh(jnp.dot(h16, wup, preferred_element_type=f32)
                          + bup_ref[...])
            hu = ((0.5 + 0.5 * tq[:, :HID]) * tq[:, HID:2 * HID]).astype(bf16)
            hp = ((0.5 + 0.5 * tq[:, 2 * HID:3 * HID]) * tq[:, 3 * HID:]).astype(bf16)
            y = (jax.lax.dot_general(wu8, hu, (((0,), (1,)), ((), ())),
                                     preferred_element_type=f32)
                 + jax.lax.dot_general(wp8, hp, (((0,), (1,)), ((), ())),
                                       preferred_element_type=f32))
            y_ref[t, :, c * cw:(c + 1) * cw] = y + bh8_ref[...]


@jax.jit
def kernel(X_seq, edge, params):
    del edge
    T, N, IN_F = X_seq.shape
    pb = params["backbone"]
    HID = pb["W_hz"].shape[0]
    f32 = jnp.float32

    wx = jnp.concatenate([pb["W_xz"], pb["W_xr"], pb["W_xh"]], axis=1)
    bx = jnp.concatenate([pb["b_xz"], pb["b_xr"], pb["b_xh"]])[None, :]
    whzr = jnp.concatenate([pb["W_hz"], pb["W_hr"]], axis=1)
    bhzr = jnp.concatenate([pb["b_hz"], pb["b_hr"]])[None, :]
    whh = pb["W_hh"]
    bhh = pb["b_hh"][None, :]

    def head_cell(p):
        w = jnp.concatenate([p["W_xz"], p["W_xh"]], axis=1)
        b = jnp.concatenate([p["b_xz"] + p["b_hz"], p["b_xh"] + p["b_hh"]])[None, :]
        return w, b

    wu, bu = head_cell(params["gru_u"])
    wp, bp = head_cell(params["gru_sp"])
    wup = jnp.concatenate([wu, wp], axis=1)
    bup = jnp.concatenate([bu, bp], axis=1)
    sxz = jnp.concatenate([jnp.full((2 * HID,), 0.5, f32),
                           jnp.ones((HID,), f32)])[None, :]
    wx = wx * sxz
    bx = bx * sxz
    whzr = whzr * 0.5
    bhzr = bhzr * 0.5
    sup = jnp.concatenate([jnp.full((HID,), -0.5, f32), jnp.ones((HID,), f32),
                           jnp.full((HID,), -0.5, f32), jnp.ones((HID,), f32)])[None, :]
    wup = wup * sup
    bup = bup * sup
    bf16 = jnp.bfloat16
    wx, whzr, whh, wup = (a.astype(bf16) for a in (wx, whzr, whh, wup))

    wu8 = jnp.zeros((HID, _HEAD_W), bf16).at[:, 0:3].set(params["W_hu"].astype(bf16))
    wp8 = (jnp.zeros((HID, _HEAD_W), bf16)
           .at[:, 3:4].set(params["W_hs"].astype(bf16))
           .at[:, 4:5].set(params["W_hp"].astype(bf16)))
    bh8 = (jnp.zeros((_HEAD_W,), f32)
           .at[0:3].set(params["b_hu"])
           .at[3].set(params["b_hs"][0])
           .at[4].set(params["b_hp"][0]))[:, None]

    bn = 2048
    xt = X_seq.transpose(0, 2, 1).astype(jnp.bfloat16)
    grid = pl.cdiv(N, bn)

    full = lambda a: pl.BlockSpec(a.shape, lambda i: (0,) * a.ndim)
    y = pl.pallas_call(
        functools.partial(_block_body, T, HID, 1),
        grid=(grid,),
        in_specs=[
            pl.BlockSpec((T, IN_F, bn), lambda i: (0, 0, i)),
            full(wx), full(bx), full(whzr), full(bhzr), full(whh), full(bhh),
            full(wup), full(bup),
            full(wu8), full(wp8), full(bh8),
        ],
        out_specs=pl.BlockSpec((T, _HEAD_W, bn), lambda i: (0, 0, i)),
        out_shape=jax.ShapeDtypeStruct((T, _HEAD_W, N), f32),
        compiler_params=pltpu.CompilerParams(
            dimension_semantics=("parallel",)),
    )(xt, wx, bx, whzr, bhzr, whh, bhh, wup, bup, wu8, wp8, bh8)

    out_u = y[:, 0:3, :].transpose(0, 2, 1)
    out_s = y[:, 3, :]
    out_p = y[:, 4, :]
    return (out_u, out_s, out_p)
